# baseline (speedup 1.0000x reference)
import functools

import jax
import jax.numpy as jnp
from jax import lax
from jax.experimental import pallas as pl
from jax.experimental.pallas import tpu as pltpu
from jax.experimental.pallas import tpu_sc as plsc


def _make_gather(BS, S, D, CB):
    info = plsc.get_sparse_core_info()
    nw = info.num_cores * info.num_subcores
    rows_per_w = BS // nw
    nchunks = rows_per_w // CB
    assert nchunks * CB == rows_per_w and nchunks % 2 == 0
    mesh = plsc.VectorSubcoreMesh(core_axis_name="c", subcore_axis_name="s")

    @functools.partial(
        pl.kernel,
        mesh=mesh,
        out_type=jax.ShapeDtypeStruct((BS, S, D), jnp.float32),
        scratch_types=[
            pltpu.VMEM((rows_per_w, S), jnp.int32),
            pltpu.VMEM((CB, S, D), jnp.float32),
            pltpu.VMEM((CB, S, D), jnp.float32),
            pltpu.SemaphoreType.DMA,
            pltpu.SemaphoreType.DMA,
            pltpu.SemaphoreType.DMA,
            pltpu.SemaphoreType.DMA,
        ],
        compiler_params=pltpu.CompilerParams(use_tc_tiling_on_sc=False),
    )
    def gather_kernel(table_hbm, x_hbm, out_hbm, idx_v, r0, r1,
                      sg0, sg1, sw0, sw1):
        wid = lax.axis_index("s") * info.num_cores + lax.axis_index("c")
        base = wid * rows_per_w
        rows = (r0, r1)
        sg = (sg0, sg1)
        sw = (sw0, sw1)

        pltpu.sync_copy(x_hbm.at[pl.ds(base, rows_per_w)], idx_v)

        def start_gather(i, b):
            for j in range(CB):
                pltpu.async_copy(table_hbm.at[idx_v.at[i * CB + j]],
                                 rows[b].at[j], sg[b])

        def wait_gather(b):
            pltpu.make_async_copy(out_hbm.at[pl.ds(base, CB)], rows[b],
                                  sg[b]).wait()

        def start_write(i, b):
            pltpu.async_copy(rows[b], out_hbm.at[pl.ds(base + i * CB, CB)],
                             sw[b])

        def wait_write(b):
            pltpu.make_async_copy(rows[b], out_hbm.at[pl.ds(base, CB)],
                                  sw[b]).wait()

        start_gather(0, 0)
        start_gather(1, 1)

        def pair(k, carry):
            for b in (0, 1):
                i = 2 * k + b
                wait_gather(b)
                start_write(i, b)

                @pl.when(i + 2 < nchunks)
                def _():
                    wait_write(b)
                    start_gather(i + 2, b)

            return carry

        lax.fori_loop(0, nchunks // 2, pair, 0)
        wait_write(0)
        wait_write(1)

    return gather_kernel


def kernel(x, table):
    bs, s = x.shape
    d = table.shape[1]
    return _make_gather(bs, s, d, 8)(table, x.astype(jnp.int32))
